# baseline (speedup 1.0000x reference)
import jax
import jax.numpy as jnp
from jax.experimental import pallas as pl
from jax.experimental.pallas import tpu as pltpu


def _head_body(g_ref, Wg_ref, bg_ref, Wf1_ref, bf1_ref, Wf2_ref, bf2_ref,
               Wo_ref, bo_ref, out_ref):
    g = jnp.maximum(jnp.dot(g_ref[...], Wg_ref[...],
                            preferred_element_type=jnp.float32) + bg_ref[...], 0.)
    g = jnp.maximum(jnp.dot(g, Wf1_ref[...],
                            preferred_element_type=jnp.float32) + bf1_ref[...], 0.)
    g = jnp.maximum(jnp.dot(g, Wf2_ref[...],
                            preferred_element_type=jnp.float32) + bf2_ref[...], 0.)
    out_ref[...] = jnp.dot(g, Wo_ref[...],
                           preferred_element_type=jnp.float32) + bo_ref[...]


def _mlp_head(g, Wg, bg, Wf1, bf1, Wf2, bf2, Wo, bo):
    return pl.pallas_call(
        _head_body,
        out_shape=jax.ShapeDtypeStruct((g.shape[0], 1), jnp.float32),
    )(g, Wg, bg.reshape(1, -1), Wf1, bf1.reshape(1, -1),
      Wf2, bf2.reshape(1, -1), Wo, bo.reshape(1, -1))


def _gat_conv(x, edge_index, W, a_src, a_dst, b, heads, out_ch):
    N = x.shape[0]
    h = (x @ W).reshape(N, heads, out_ch)
    loop = jnp.arange(N, dtype=edge_index.dtype)
    src = jnp.concatenate([edge_index[0], loop])
    dst = jnp.concatenate([edge_index[1], loop])
    asrc = (h * a_src[None, :, :]).sum(-1)
    adst = (h * a_dst[None, :, :]).sum(-1)
    alpha = jax.nn.leaky_relu(asrc[src] + adst[dst], negative_slope=0.2)
    amax = jax.ops.segment_max(alpha, dst, num_segments=N)
    amax = jnp.where(jnp.isfinite(amax), amax, 0.0)
    e = jnp.exp(alpha - amax[dst])
    denom = jax.ops.segment_sum(e, dst, num_segments=N)
    coef = e / (denom[dst] + 1e-16)
    msg = h[src] * coef[:, :, None]
    out = jax.ops.segment_sum(msg, dst, num_segments=N)
    return out.reshape(N, heads * out_ch) + b


def kernel(x, edge_index, batch, W1, a_src1, a_dst1, b1, W2, a_src2, a_dst2,
           b2, Wg, bg, Wf1, bf1, Wf2, bf2, Wo, bo):
    h = jax.nn.elu(_gat_conv(x, edge_index, W1, a_src1, a_dst1, b1, 10, 78))
    h = jax.nn.relu(_gat_conv(h, edge_index, W2, a_src2, a_dst2, b2, 1, 128))
    g = jax.ops.segment_max(h, batch, num_segments=128)
    g = jnp.where(jnp.isfinite(g), g, 0.0)
    return _mlp_head(g, Wg, bg, Wf1, bf1, Wf2, bf2, Wo, bo)


# speedup vs baseline: 10.3436x; 10.3436x over previous
import functools

import jax
import jax.numpy as jnp
from jax import lax
from jax.experimental import pallas as pl
from jax.experimental.pallas import tpu as pltpu
from jax.experimental.pallas import tpu_sc as plsc

_NC, _NS = 2, 16
_NT = _NC * _NS
_BLK = 128
_N = 10000
_NPAD = 10240
_RPS = _NPAD // _NS
_E = 170000
_NBLK = -(-_E // (_NT * _BLK))
_EPAD = _NT * _BLK * _NBLK


def _make_edge_pass(width, nheads, nblk):
    head_w = width // nheads
    mesh = plsc.VectorSubcoreMesh(core_axis_name="c", subcore_axis_name="s")
    scratch = [
        pltpu.VMEM((_BLK,), jnp.int32),
        pltpu.VMEM((_BLK,), jnp.int32),
        pltpu.VMEM((_BLK, width), jnp.float32),
        pltpu.VMEM((_BLK, 16), jnp.float32),
        pltpu.VMEM((_BLK, 16), jnp.float32),
        pltpu.VMEM_SHARED((_NPAD, width), jnp.float32),
        pltpu.SemaphoreType.DMA,
        pltpu.SemaphoreType.DMA,
        pltpu.SemaphoreType.DMA,
    ]

    @functools.partial(
        pl.kernel, mesh=mesh,
        out_type=jax.ShapeDtypeStruct((_NC * _NPAD, width), jnp.float32),
        scratch_types=scratch,
        compiler_params=pltpu.CompilerParams(use_tc_tiling_on_sc=False))
    def kern(table_h, srcs_h, dsts_h, asrc_h, adst_h, out_h, *sc):
        src_v, dst_v, rows_v, as_v, ad_v = sc[0], sc[1], sc[2], sc[3], sc[4]
        accum, sem, sem2, sem3 = sc[5], sc[6], sc[7], sc[8]
        cid = lax.axis_index("c")
        sid = lax.axis_index("s")
        wid = sid * _NC + cid

        def zrow(i, carry):
            for c in range(width // 16):
                rows_v[i, pl.ds(c * 16, 16)] = jnp.zeros((16,), jnp.float32)
            return carry
        lax.fori_loop(0, _BLK, zrow, 0)
        for k in range(_RPS // _BLK):
            pltpu.sync_copy(rows_v,
                            accum.at[pl.ds(sid * _RPS + k * _BLK, _BLK)])
        plsc.subcore_barrier()

        def block(b, carry):
            base = (wid * nblk + b) * _BLK
            pltpu.sync_copy(srcs_h.at[pl.ds(base, _BLK)], src_v)
            pltpu.sync_copy(dsts_h.at[pl.ds(base, _BLK)], dst_v)
            cp1 = pltpu.async_copy(table_h.at[src_v], rows_v, sem)
            cp2 = pltpu.async_copy(asrc_h.at[src_v], as_v, sem2)
            cp3 = pltpu.async_copy(adst_h.at[dst_v], ad_v, sem3)
            cp1.wait()
            cp2.wait()
            cp3.wait()
            def edge(i, carry2):
                a = as_v[i, pl.ds(0, 16)] + ad_v[i, pl.ds(0, 16)]
                a = jnp.where(a > 0, a, 0.2 * a)
                w16 = jnp.exp(a)
                for h in range(nheads):
                    wv = jnp.full((16,), w16[h], jnp.float32)
                    for c in range(head_w // 16):
                        col = h * head_w + c * 16
                        rows_v[i, pl.ds(col, 16)] = (
                            rows_v[i, pl.ds(col, 16)] * wv)
                return carry2
            lax.fori_loop(0, _BLK, edge, 0)
            pltpu.sync_copy(rows_v, accum.at[dst_v], add=True)
            return carry
        lax.fori_loop(0, nblk, block, 0)

        plsc.subcore_barrier()
        for k in range(_RPS // _BLK):
            off = sid * _RPS + k * _BLK
            pltpu.sync_copy(accum.at[pl.ds(off, _BLK)],
                            out_h.at[pl.ds(cid * _NPAD + off, _BLK)])

    return kern


def _tc1_body(x_ref, W1_ref, As_ref, Ad_ref, h_ref, as_ref, ad_ref):
    h = jnp.dot(x_ref[...], W1_ref[...], preferred_element_type=jnp.float32)
    h_ref[...] = h
    as_ref[...] = jnp.dot(h, As_ref[...], preferred_element_type=jnp.float32)
    ad_ref[...] = jnp.dot(h, Ad_ref[...], preferred_element_type=jnp.float32)


def _tc2_body(p0_ref, p1_ref, b1_ref, W2_ref, A2_ref, h2_ref, a2_ref):
    acc = p0_ref[...] + p1_ref[...]
    pieces = []
    for t in range(10):
        base = t * 80
        feat = acc[:, base:base + 78]
        den = acc[:, base + 78:base + 79]
        pieces.append(feat / (den + 1e-16))
    hpre = jnp.concatenate(pieces, axis=1) + b1_ref[...]
    h = jnp.where(hpre > 0, hpre, jnp.exp(jnp.minimum(hpre, 0.0)) - 1.0)
    h2 = jnp.dot(h, W2_ref[...], preferred_element_type=jnp.float32)
    h2_ref[...] = h2
    a2_ref[...] = jnp.dot(h2, A2_ref[...], preferred_element_type=jnp.float32)


def _fin_body(p0_ref, p1_ref, b2_ref, h_ref):
    acc = p0_ref[...] + p1_ref[...]
    h = acc[:, :128] / (acc[:, 128:129] + 1e-16) + b2_ref[...]
    h_ref[...] = jnp.maximum(h, 0.0)


def _pool_body(ht_ref, mt_ref, out_ref):
    ht = ht_ref[...]

    def g(b, carry):
        bias = mt_ref[pl.ds(b, 1), :]
        red = jnp.max(ht + bias, axis=1)
        out_ref[pl.ds(b, 1), :] = red.reshape(1, 128)
        return carry
    lax.fori_loop(0, 128, g, 0)


def _head_body(g_ref, Wg_ref, bg_ref, Wf1_ref, bf1_ref, Wf2_ref, bf2_ref,
               Wo_ref, bo_ref, out_ref):
    g = jnp.maximum(g_ref[...], 0.0)
    g = jnp.maximum(jnp.dot(g, Wg_ref[...],
                            preferred_element_type=jnp.float32) + bg_ref[...], 0.)
    g = jnp.maximum(jnp.dot(g, Wf1_ref[...],
                            preferred_element_type=jnp.float32) + bf1_ref[...], 0.)
    g = jnp.maximum(jnp.dot(g, Wf2_ref[...],
                            preferred_element_type=jnp.float32) + bf2_ref[...], 0.)
    out_ref[...] = jnp.dot(g, Wo_ref[...],
                           preferred_element_type=jnp.float32) + bo_ref[...]


def kernel(x, edge_index, batch, W1, a_src1, a_dst1, b1, W2, a_src2, a_dst2,
           b2, Wg, bg, Wf1, bf1, Wf2, bf2, Wo, bo):
    f32 = jnp.float32
    loop = jnp.arange(_N, dtype=jnp.int32)
    srcs = jnp.concatenate([edge_index[0].astype(jnp.int32), loop,
                            jnp.zeros((_EPAD - _E,), jnp.int32)])
    dsts = jnp.concatenate([edge_index[1].astype(jnp.int32), loop,
                            jnp.full((_EPAD - _E,), _N, jnp.int32)])

    rows780 = jnp.arange(780)
    As1 = jnp.zeros((780, 10), f32).at[rows780, rows780 // 78].set(
        a_src1.reshape(-1))
    Ad1 = jnp.zeros((780, 10), f32).at[rows780, rows780 // 78].set(
        a_dst1.reshape(-1))
    h1, as1, ad1 = pl.pallas_call(
        _tc1_body,
        grid=(5,),
        in_specs=[pl.BlockSpec((2000, 78), lambda i: (i, 0)),
                  pl.BlockSpec((78, 780), lambda i: (0, 0)),
                  pl.BlockSpec((780, 10), lambda i: (0, 0)),
                  pl.BlockSpec((780, 10), lambda i: (0, 0))],
        out_specs=[pl.BlockSpec((2000, 780), lambda i: (i, 0)),
                   pl.BlockSpec((2000, 10), lambda i: (i, 0)),
                   pl.BlockSpec((2000, 10), lambda i: (i, 0))],
        out_shape=[jax.ShapeDtypeStruct((_N, 780), f32),
                   jax.ShapeDtypeStruct((_N, 10), f32),
                   jax.ShapeDtypeStruct((_N, 10), f32)],
    )(x, W1, As1, Ad1)

    ones = jnp.ones((_N, 1), f32)
    z1 = jnp.zeros((_N, 1), f32)
    ep1 = _make_edge_pass(160, 2, _NBLK)
    parts = []
    for g in range(5):
        tbl = jnp.concatenate(
            [h1[:, 156 * g:156 * g + 78], ones, z1,
             h1[:, 156 * g + 78:156 * g + 156], ones, z1], axis=1)
        asg = jnp.pad(as1[:, 2 * g:2 * g + 2], ((0, _NPAD - _N), (0, 14)))
        adg = jnp.pad(ad1[:, 2 * g:2 * g + 2], ((0, _NPAD - _N), (0, 14)))
        parts.append(ep1(tbl, srcs, dsts, asg, adg))
    p0 = jnp.concatenate([p[:_N] for p in parts], axis=1)
    p1 = jnp.concatenate([p[_NPAD:_NPAD + _N] for p in parts], axis=1)

    A2 = jnp.concatenate([a_src2.T, a_dst2.T], axis=1)
    h2, a2 = pl.pallas_call(
        _tc2_body,
        grid=(5,),
        in_specs=[pl.BlockSpec((2000, 800), lambda i: (i, 0)),
                  pl.BlockSpec((2000, 800), lambda i: (i, 0)),
                  pl.BlockSpec((1, 780), lambda i: (0, 0)),
                  pl.BlockSpec((780, 128), lambda i: (0, 0)),
                  pl.BlockSpec((128, 2), lambda i: (0, 0))],
        out_specs=[pl.BlockSpec((2000, 128), lambda i: (i, 0)),
                   pl.BlockSpec((2000, 2), lambda i: (i, 0))],
        out_shape=[jax.ShapeDtypeStruct((_N, 128), f32),
                   jax.ShapeDtypeStruct((_N, 2), f32)],
    )(p0, p1, b1.reshape(1, 780), W2, A2)

    tbl2 = jnp.concatenate([h2, ones, jnp.zeros((_N, 15), f32)], axis=1)
    as2p = jnp.pad(a2[:, 0:1], ((0, _NPAD - _N), (0, 15)))
    ad2p = jnp.pad(a2[:, 1:2], ((0, _NPAD - _N), (0, 15)))
    ep2 = _make_edge_pass(144, 1, _NBLK)
    part2 = ep2(tbl2, srcs, dsts, as2p, ad2p)

    hf = pl.pallas_call(
        _fin_body,
        out_shape=jax.ShapeDtypeStruct((_N, 128), f32),
    )(part2[:_N], part2[_NPAD:_NPAD + _N], b2.reshape(1, 128))

    mt = jnp.where(jnp.arange(128)[:, None] == batch[None, :], 0.0, -1e30
                   ).astype(f32)
    pooled = pl.pallas_call(
        _pool_body,
        out_shape=jax.ShapeDtypeStruct((128, 128), f32),
    )(hf.T, mt)

    return pl.pallas_call(
        _head_body,
        out_shape=jax.ShapeDtypeStruct((128, 1), f32),
    )(pooled, Wg, bg.reshape(1, -1), Wf1, bf1.reshape(1, -1),
      Wf2, bf2.reshape(1, -1), Wo, bo.reshape(1, -1))
